# baseline (speedup 1.0000x reference)
import functools
import math

import jax
import jax.numpy as jnp
from jax import lax
from jax.experimental import pallas as pl
from jax.experimental.pallas import tpu as pltpu
from jax.experimental.pallas import tpu_sc as plsc

B, N = 32, 32
NUM_BASIS = 20
R_MAX = 10.0
D_EMB = 32
D_SCAL = 64
_C = 1.0 / math.sqrt(D_EMB)
_CK = [R_MAX / (NUM_BASIS - 1) * k for k in range(NUM_BASIS)]
NC, NS, L = 2, 16, 16


def _fold_body(ae_exp_ref, fc2w1_ref, fc2b1_ref, wself_ref, wread_ref,
               atom_ref, vt_ref, w3_ref):
    v = jnp.dot(ae_exp_ref[...], fc2w1_ref[...])
    s0 = jnp.dot(ae_exp_ref[...], fc2b1_ref[...])
    q = jnp.dot(wself_ref[...], wread_ref[...])
    aeq = jnp.dot(atom_ref[...], q) * _C
    vt_ref[...] = v
    w3_ref[...] = (_C * (N - 1)) * s0 + aeq


def _zchunk(nav, c):
    a0 = nav[0, pl.ds(L * c, L)]
    a1 = nav[1, pl.ds(L * c, L)]
    a2 = nav[2, pl.ds(L * c, L)]
    one = jnp.full((L,), 1, jnp.int32)
    z = jnp.where(a1 > a0, one, jnp.zeros((L,), jnp.int32))
    z = jnp.where(a2 > jnp.maximum(a0, a1), one + one, z)
    return z


def _rsqrt_newton(x):
    i = lax.bitcast_convert_type(x, jnp.int32)
    y = lax.bitcast_convert_type(0x5F3759DF - (i >> 1), jnp.float32)
    for _ in range(4):
        y = y * (1.5 - 0.5 * x * y * y)
    return y


def _sc_body(posT, naT, vt, w3p, fc1w, fc1b, gpad, out,
             posv, nav, vtv, w3v, fc1v, fc1bv, gv, outv):
    wid = lax.axis_index("s") * NC + lax.axis_index("c")
    pltpu.sync_copy(posT.at[wid], posv)
    pltpu.sync_copy(naT.at[wid], nav)
    pltpu.sync_copy(vt, vtv)
    pltpu.sync_copy(w3p, w3v)
    pltpu.sync_copy(fc1w, fc1v)
    pltpu.sync_copy(fc1b, fc1bv)
    pltpu.sync_copy(gpad, gv)

    g = gv[...][0]
    zero_i = jnp.zeros((L,), jnp.int32)
    blo = fc1bv[pl.ds(0, L)]
    bhi = fc1bv[pl.ds(D_EMB - L, L)]
    total = jnp.zeros((L,), jnp.float32)
    for c in range(N // L):
        z = _zchunk(nav, c)
        total = total + plsc.load_gather(w3v, [z, lax.iota(jnp.int32, L)])
        pjx = posv[0, pl.ds(L * c, L)]
        pjy = posv[1, pl.ds(L * c, L)]
        pjz = posv[2, pl.ds(L * c, L)]
        j_ids = lax.iota(jnp.int32, L) + (L * c)

        def body(i, pacc, z=z, pjx=pjx, pjy=pjy, pjz=pjz, j_ids=j_ids):
            isplat = jnp.full((L,), i, jnp.int32)
            dx = plsc.load_gather(posv, [zero_i, isplat]) - pjx
            dy = plsc.load_gather(posv, [zero_i + 1, isplat]) - pjy
            dz = plsc.load_gather(posv, [zero_i + 2, isplat]) - pjz
            r2 = jnp.maximum(dx * dx + dy * dy + dz * dz, 1e-20)
            d = r2 * _rsqrt_newton(r2)
            rbfs = []
            for k in range(NUM_BASIS):
                t = d - _CK[k]
                rbfs.append(jnp.exp(-g * t * t))
            acc = jnp.zeros((L,), jnp.float32)
            for u in range(D_EMB):
                row_lo = fc1v[u, pl.ds(0, L)]
                row_hi = fc1v[u, pl.ds(NUM_BASIS - L, L)]
                bias = blo[u] if u < L else bhi[u - (D_EMB - L)]
                hv = jnp.broadcast_to(bias, (L,))
                for k in range(NUM_BASIS):
                    w = row_lo[k] if k < L else row_hi[k - (NUM_BASIS - L)]
                    hv = hv + w * rbfs[k]
                hv = jnp.maximum(hv, 0.0)
                vzu = plsc.load_gather(vtv, [z, jnp.full((L,), u, jnp.int32)])
                acc = acc + hv * vzu
            return pacc + jnp.where(j_ids != i, acc, 0.0)

        pair = lax.fori_loop(0, N, body, jnp.zeros((L,), jnp.float32))
        total = total + _C * pair

    tot = jnp.sum(0.125 * total)
    outv[...] = jnp.broadcast_to(tot, (L,))
    pltpu.sync_copy(outv, out.at[wid])


def kernel(pos, node_attrs, atom_emb, gamma, fc1_w, fc1_b, fc2_w, fc2_b,
           w_self, w_readout):
    ae_exp = (atom_emb[:, :, None] * w_readout[None, None, :, 0]).reshape(
        3, D_EMB * D_SCAL)
    fc2w1 = fc2_w[:D_EMB * D_SCAL, :]
    fc2b1 = fc2_b[:D_EMB * D_SCAL].reshape(D_EMB * D_SCAL, 1)

    vt, w3 = pl.pallas_call(
        _fold_body,
        out_shape=(jax.ShapeDtypeStruct((3, 32), jnp.float32),
                   jax.ShapeDtypeStruct((3, 1), jnp.float32)),
    )(ae_exp, fc2w1, fc2b1, w_self, w_readout, atom_emb)

    posT = pos.transpose(0, 2, 1)
    naT = node_attrs.transpose(0, 2, 1)
    w3p = jnp.broadcast_to(w3, (3, L))
    gpad = jnp.broadcast_to(jnp.asarray(gamma, jnp.float32), (L,))

    mesh = plsc.VectorSubcoreMesh(core_axis_name="c", subcore_axis_name="s")
    sc = functools.partial(
        pl.kernel,
        mesh=mesh,
        compiler_params=pltpu.CompilerParams(needs_layout_passes=False),
        out_type=jax.ShapeDtypeStruct((B, L), jnp.float32),
        scratch_types=[
            pltpu.VMEM((3, N), jnp.float32),
            pltpu.VMEM((3, N), jnp.float32),
            pltpu.VMEM((3, 32), jnp.float32),
            pltpu.VMEM((3, L), jnp.float32),
            pltpu.VMEM((D_EMB, NUM_BASIS), jnp.float32),
            pltpu.VMEM((D_EMB,), jnp.float32),
            pltpu.VMEM((L,), jnp.float32),
            pltpu.VMEM((L,), jnp.float32),
        ],
    )(_sc_body)
    out = sc(posT, naT, vt, w3p, fc1_w, fc1_b, gpad)
    return out[:, 0:1]


# speedup vs baseline: 1.1091x; 1.1091x over previous
import functools
import math

import jax
import jax.numpy as jnp
from jax import lax
from jax.experimental import pallas as pl
from jax.experimental.pallas import tpu as pltpu
from jax.experimental.pallas import tpu_sc as plsc

B, N = 32, 32
NUM_BASIS = 20
R_MAX = 10.0
D_EMB = 32
D_SCAL = 64
_C = 1.0 / math.sqrt(D_EMB)
NC, NS, L = 2, 16, 16
BPS = 4
P = BPS * N * N


def _tc_body(pos_ref, fc1wT_ref, fc1b_ref, gamma_ref, ae_exp_ref, fc2w1_ref,
             fc2b1_ref, wself_ref, wread_ref, atom_ref,
             hid_ref, vt_ref, w3_ref):
    v = jnp.dot(ae_exp_ref[...], fc2w1_ref[...])
    s0 = jnp.dot(ae_exp_ref[...], fc2b1_ref[...])
    q = jnp.dot(wself_ref[...], wread_ref[...])
    aeq = jnp.dot(atom_ref[...], q) * _C
    vt_ref[...] = v
    w3_ref[...] = (_C * (N - 1)) * s0 + aeq

    pos = pos_ref[...]
    pi = jnp.broadcast_to(pos[:, :, None, :], (BPS, N, N, 3)).reshape(P, 3)
    pj = jnp.broadcast_to(pos[:, None, :, :], (BPS, N, N, 3)).reshape(P, 3)
    diff = pi - pj
    d2 = jnp.sum(diff * diff, axis=1, keepdims=True)
    d = jnp.sqrt(jnp.maximum(d2, 0.0))
    centers = jax.lax.broadcasted_iota(jnp.int32, (1, NUM_BASIS), 1).astype(
        jnp.float32) * (R_MAX / (NUM_BASIS - 1))
    g = gamma_ref[...]
    t = d - centers
    rbf = jnp.exp(-g * t * t)
    hid_ref[...] = jnp.maximum(
        jnp.dot(rbf, fc1wT_ref[...]) + fc1b_ref[...], 0.0)


def _zchunk(nav, c):
    a0 = nav[0, pl.ds(L * c, L)]
    a1 = nav[1, pl.ds(L * c, L)]
    a2 = nav[2, pl.ds(L * c, L)]
    one = jnp.full((L,), 1, jnp.int32)
    z = jnp.where(a1 > a0, one, jnp.zeros((L,), jnp.int32))
    z = jnp.where(a2 > jnp.maximum(a0, a1), one + one, z)
    return z


def _sc_body(hid3, naT, vt, w3p, out, hidv, nav, vtv, w3v, outv):
    wid = lax.axis_index("s") * NC + lax.axis_index("c")
    pltpu.sync_copy(hid3.at[wid], hidv)
    pltpu.sync_copy(naT.at[wid], nav)
    pltpu.sync_copy(vt, vtv)
    pltpu.sync_copy(w3p, w3v)

    total = jnp.zeros((L,), jnp.float32)
    for c in range(N // L):
        z = _zchunk(nav, c)
        total = total + plsc.load_gather(w3v, [z, lax.iota(jnp.int32, L)])
        j_ids = lax.iota(jnp.int32, L) + (L * c)
        jbase = j_ids * D_EMB
        vzs = [plsc.load_gather(vtv, [z, jnp.full((L,), u, jnp.int32)])
               for u in range(D_EMB)]

        def body(i, pacc, vzs=vzs, j_ids=j_ids, jbase=jbase):
            ebase = jbase + i * (N * D_EMB)
            acc = jnp.zeros((L,), jnp.float32)
            for u in range(D_EMB):
                hu = plsc.load_gather(hidv, [ebase + u])
                acc = acc + hu * vzs[u]
            return pacc + jnp.where(j_ids != i, acc, 0.0)

        pair = lax.fori_loop(0, N, body, jnp.zeros((L,), jnp.float32))
        total = total + _C * pair

    tot = jnp.sum(0.125 * total)
    outv[...] = jnp.broadcast_to(tot, (L,))
    pltpu.sync_copy(outv, out.at[wid])


def kernel(pos, node_attrs, atom_emb, gamma, fc1_w, fc1_b, fc2_w, fc2_b,
           w_self, w_readout):
    ae_exp = (atom_emb[:, :, None] * w_readout[None, None, :, 0]).reshape(
        3, D_EMB * D_SCAL)
    fc2w1 = fc2_w[:D_EMB * D_SCAL, :]
    fc2b1 = fc2_b[:D_EMB * D_SCAL].reshape(D_EMB * D_SCAL, 1)
    fc1wT = fc1_w.T
    fc1b = fc1_b.reshape(1, 32)
    gamma2 = jnp.asarray(gamma, jnp.float32).reshape(1, 1)

    grid = (B // BPS,)
    full = lambda shape: pl.BlockSpec(shape, lambda b: (0,) * len(shape))
    hid, vt, w3 = pl.pallas_call(
        _tc_body,
        grid=grid,
        in_specs=[
            pl.BlockSpec((BPS, N, 3), lambda b: (b, 0, 0)),
            full((NUM_BASIS, 32)),
            full((1, 32)),
            full((1, 1)),
            full((3, D_EMB * D_SCAL)),
            full((D_EMB * D_SCAL, 32)),
            full((D_EMB * D_SCAL, 1)),
            full((D_EMB, D_SCAL)),
            full((D_SCAL, 1)),
            full((3, D_EMB)),
        ],
        out_specs=[
            pl.BlockSpec((P, D_EMB), lambda b: (b, 0)),
            full((3, 32)),
            full((3, 1)),
        ],
        out_shape=[
            jax.ShapeDtypeStruct((B * N * N, D_EMB), jnp.float32),
            jax.ShapeDtypeStruct((3, 32), jnp.float32),
            jax.ShapeDtypeStruct((3, 1), jnp.float32),
        ],
    )(pos, fc1wT, fc1b, gamma2, ae_exp, fc2w1, fc2b1, w_self, w_readout,
      atom_emb)

    naT = node_attrs.transpose(0, 2, 1)
    w3p = jnp.broadcast_to(w3, (3, L))
    hid3 = hid.reshape(B, N * N * D_EMB)

    mesh = plsc.VectorSubcoreMesh(core_axis_name="c", subcore_axis_name="s")
    sc = functools.partial(
        pl.kernel,
        mesh=mesh,
        compiler_params=pltpu.CompilerParams(needs_layout_passes=False),
        out_type=jax.ShapeDtypeStruct((B, L), jnp.float32),
        scratch_types=[
            pltpu.VMEM((N * N * D_EMB,), jnp.float32),
            pltpu.VMEM((3, N), jnp.float32),
            pltpu.VMEM((3, 32), jnp.float32),
            pltpu.VMEM((3, L), jnp.float32),
            pltpu.VMEM((L,), jnp.float32),
        ],
    )(_sc_body)
    out = sc(hid3, naT, vt, w3p)
    return out[:, 0:1]


# speedup vs baseline: 1.5329x; 1.3821x over previous
import functools
import math

import jax
import jax.numpy as jnp
from jax import lax
from jax.experimental import pallas as pl
from jax.experimental.pallas import tpu as pltpu
from jax.experimental.pallas import tpu_sc as plsc

B, N = 32, 32
NUM_BASIS = 20
R_MAX = 10.0
D_EMB = 32
D_SCAL = 64
_C = 1.0 / math.sqrt(D_EMB)
NC, NS, L = 2, 16, 16
BPS = 4
P = BPS * N * N


def _tc_body(pos_ref, fc1wT_ref, fc1b_ref, gamma_ref, ae_exp_ref, fc2w1_ref,
             fc2b1_ref, wself_ref, wread_ref, atom_ref,
             hid_ref, vt_ref, w3_ref):
    v = jnp.dot(ae_exp_ref[...], fc2w1_ref[...])
    s0 = jnp.dot(ae_exp_ref[...], fc2b1_ref[...])
    q = jnp.dot(wself_ref[...], wread_ref[...])
    aeq = jnp.dot(atom_ref[...], q) * _C
    vt_ref[...] = v
    w3_ref[...] = (_C * (N - 1)) * s0 + aeq

    pos = pos_ref[...]
    pi = jnp.broadcast_to(pos[:, :, None, :], (BPS, N, N, 3)).reshape(P, 3)
    pj = jnp.broadcast_to(pos[:, None, :, :], (BPS, N, N, 3)).reshape(P, 3)
    diff = pi - pj
    d2 = jnp.sum(diff * diff, axis=1, keepdims=True)
    d = jnp.sqrt(jnp.maximum(d2, 0.0))
    centers = jax.lax.broadcasted_iota(jnp.int32, (1, NUM_BASIS), 1).astype(
        jnp.float32) * (R_MAX / (NUM_BASIS - 1))
    g = gamma_ref[...]
    t = d - centers
    rbf = jnp.exp(-g * t * t)
    hid = jnp.maximum(jnp.dot(rbf, fc1wT_ref[...]) + fc1b_ref[...], 0.0)
    hid_ref[...] = jnp.transpose(hid.reshape(BPS, N * N, D_EMB), (0, 2, 1))


def _zchunk(nav, c):
    a0 = nav[0, pl.ds(L * c, L)]
    a1 = nav[1, pl.ds(L * c, L)]
    a2 = nav[2, pl.ds(L * c, L)]
    one = jnp.full((L,), 1, jnp.int32)
    z = jnp.where(a1 > a0, one, jnp.zeros((L,), jnp.int32))
    z = jnp.where(a2 > jnp.maximum(a0, a1), one + one, z)
    return z


def _sc_body(hid3, naT, vt, w3p, out, hidv, nav, vtv, w3v, outv):
    wid = lax.axis_index("s") * NC + lax.axis_index("c")
    pltpu.sync_copy(hid3.at[wid], hidv)
    pltpu.sync_copy(naT.at[wid], nav)
    pltpu.sync_copy(vt, vtv)
    pltpu.sync_copy(w3p, w3v)

    total = jnp.zeros((L,), jnp.float32)
    for c in range(N // L):
        z = _zchunk(nav, c)
        total = total + plsc.load_gather(w3v, [z, lax.iota(jnp.int32, L)])
        j_ids = lax.iota(jnp.int32, L) + (L * c)
        vzs = [plsc.load_gather(vtv, [z, jnp.full((L,), u, jnp.int32)])
               for u in range(D_EMB)]

        def body(i, pacc, vzs=vzs, j_ids=j_ids, c=c):
            out = pacc
            for half in range(2):
                i2 = i + half * (N // 2)
                accs = [jnp.zeros((L,), jnp.float32) for _ in range(4)]
                for u in range(D_EMB):
                    hu = hidv[u, pl.ds(i2 * N + L * c, L)]
                    accs[u % 4] = accs[u % 4] + hu * vzs[u]
                acc = (accs[0] + accs[1]) + (accs[2] + accs[3])
                out = out + jnp.where(j_ids != i2, acc, 0.0)
            return out

        pair = lax.fori_loop(0, N // 2, body, jnp.zeros((L,), jnp.float32))
        total = total + _C * pair

    tot = jnp.sum(0.125 * total)
    outv[...] = jnp.broadcast_to(tot, (L,))
    pltpu.sync_copy(outv, out.at[wid])


def kernel(pos, node_attrs, atom_emb, gamma, fc1_w, fc1_b, fc2_w, fc2_b,
           w_self, w_readout):
    ae_exp = (atom_emb[:, :, None] * w_readout[None, None, :, 0]).reshape(
        3, D_EMB * D_SCAL)
    fc2w1 = fc2_w[:D_EMB * D_SCAL, :]
    fc2b1 = fc2_b[:D_EMB * D_SCAL].reshape(D_EMB * D_SCAL, 1)
    fc1wT = fc1_w.T
    fc1b = fc1_b.reshape(1, 32)
    gamma2 = jnp.asarray(gamma, jnp.float32).reshape(1, 1)

    grid = (B // BPS,)
    full = lambda shape: pl.BlockSpec(shape, lambda b: (0,) * len(shape))
    hid, vt, w3 = pl.pallas_call(
        _tc_body,
        grid=grid,
        in_specs=[
            pl.BlockSpec((BPS, N, 3), lambda b: (b, 0, 0)),
            full((NUM_BASIS, 32)),
            full((1, 32)),
            full((1, 1)),
            full((3, D_EMB * D_SCAL)),
            full((D_EMB * D_SCAL, 32)),
            full((D_EMB * D_SCAL, 1)),
            full((D_EMB, D_SCAL)),
            full((D_SCAL, 1)),
            full((3, D_EMB)),
        ],
        out_specs=[
            pl.BlockSpec((BPS, D_EMB, N * N), lambda b: (b, 0, 0)),
            full((3, 32)),
            full((3, 1)),
        ],
        out_shape=[
            jax.ShapeDtypeStruct((B, D_EMB, N * N), jnp.float32),
            jax.ShapeDtypeStruct((3, 32), jnp.float32),
            jax.ShapeDtypeStruct((3, 1), jnp.float32),
        ],
    )(pos, fc1wT, fc1b, gamma2, ae_exp, fc2w1, fc2b1, w_self, w_readout,
      atom_emb)

    naT = node_attrs.transpose(0, 2, 1)
    w3p = jnp.broadcast_to(w3, (3, L))
    hid3 = hid

    mesh = plsc.VectorSubcoreMesh(core_axis_name="c", subcore_axis_name="s")
    sc = functools.partial(
        pl.kernel,
        mesh=mesh,
        compiler_params=pltpu.CompilerParams(needs_layout_passes=False),
        out_type=jax.ShapeDtypeStruct((B, L), jnp.float32),
        scratch_types=[
            pltpu.VMEM((D_EMB, N * N), jnp.float32),
            pltpu.VMEM((3, N), jnp.float32),
            pltpu.VMEM((3, 32), jnp.float32),
            pltpu.VMEM((3, L), jnp.float32),
            pltpu.VMEM((L,), jnp.float32),
        ],
    )(_sc_body)
    out = sc(hid3, naT, vt, w3p)
    return out[:, 0:1]


# speedup vs baseline: 1.5492x; 1.0106x over previous
import functools
import math

import jax
import jax.numpy as jnp
from jax import lax
from jax.experimental import pallas as pl
from jax.experimental.pallas import tpu as pltpu
from jax.experimental.pallas import tpu_sc as plsc

B, N = 32, 32
NUM_BASIS = 20
R_MAX = 10.0
D_EMB = 32
D_SCAL = 64
_C = 1.0 / math.sqrt(D_EMB)
NC, NS, L = 2, 16, 16
BPS = 8
P = BPS * N * N


def _tc_body(pos_ref, fc1wT_ref, fc1b_ref, gamma_ref, ae_exp_ref, fc2w1_ref,
             fc2b1_ref, wself_ref, wread_ref, atom_ref,
             hid_ref, vt_ref):
    @pl.when(pl.program_id(0) == 0)
    def _():
        v = jnp.dot(ae_exp_ref[...], fc2w1_ref[...])
        s0 = jnp.dot(ae_exp_ref[...], fc2b1_ref[...])
        q = jnp.dot(wself_ref[...], wread_ref[...])
        aeq = jnp.dot(atom_ref[...], q) * _C
        w3 = (_C * (N - 1)) * s0 + aeq
        vt_ref[...] = jnp.concatenate([v, w3], axis=1)

    pos = pos_ref[...]
    pi = jnp.broadcast_to(pos[:, :, None, :], (BPS, N, N, 3)).reshape(P, 3)
    pj = jnp.broadcast_to(pos[:, None, :, :], (BPS, N, N, 3)).reshape(P, 3)
    diff = pi - pj
    d2 = jnp.sum(diff * diff, axis=1, keepdims=True)
    d = jnp.sqrt(jnp.maximum(d2, 0.0))
    centers = jax.lax.broadcasted_iota(jnp.int32, (1, NUM_BASIS), 1).astype(
        jnp.float32) * (R_MAX / (NUM_BASIS - 1))
    g = gamma_ref[...]
    t = d - centers
    rbf = jnp.exp(-g * t * t)
    hid = jnp.maximum(jnp.dot(rbf, fc1wT_ref[...]) + fc1b_ref[...], 0.0)
    hid_ref[...] = jnp.transpose(hid.reshape(BPS, N * N, D_EMB), (0, 2, 1))


def _zchunk(nav, c):
    a0 = nav[0, pl.ds(L * c, L)]
    a1 = nav[1, pl.ds(L * c, L)]
    a2 = nav[2, pl.ds(L * c, L)]
    one = jnp.full((L,), 1, jnp.int32)
    z = jnp.where(a1 > a0, one, jnp.zeros((L,), jnp.int32))
    z = jnp.where(a2 > jnp.maximum(a0, a1), one + one, z)
    return z


def _zchunk2(nav, c):
    jr = lax.iota(jnp.int32, L) + (L * c)
    a0 = plsc.load_gather(nav, [jr, jnp.zeros((L,), jnp.int32)])
    a1 = plsc.load_gather(nav, [jr, jnp.zeros((L,), jnp.int32) + 1])
    a2 = plsc.load_gather(nav, [jr, jnp.zeros((L,), jnp.int32) + 2])
    one = jnp.full((L,), 1, jnp.int32)
    z = jnp.where(a1 > a0, one, jnp.zeros((L,), jnp.int32))
    z = jnp.where(a2 > jnp.maximum(a0, a1), one + one, z)
    return z


def _sc_body(hid3, na, vt, out, hidv, nav, vtv, outv):
    wid = lax.axis_index("s") * NC + lax.axis_index("c")
    pltpu.sync_copy(hid3.at[wid], hidv)
    pltpu.sync_copy(na.at[wid], nav)
    pltpu.sync_copy(vt, vtv)

    total = jnp.zeros((L,), jnp.float32)
    for c in range(N // L):
        z = _zchunk2(nav, c)
        total = total + plsc.load_gather(
            vtv, [z, jnp.full((L,), D_EMB, jnp.int32)])
        j_ids = lax.iota(jnp.int32, L) + (L * c)
        vzs = [plsc.load_gather(vtv, [z, jnp.full((L,), u, jnp.int32)])
               for u in range(D_EMB)]

        def body(i, pacc, vzs=vzs, j_ids=j_ids, c=c):
            out = pacc
            for half in range(2):
                i2 = i + half * (N // 2)
                accs = [jnp.zeros((L,), jnp.float32) for _ in range(4)]
                for u in range(D_EMB):
                    hu = hidv[u, pl.ds(i2 * N + L * c, L)]
                    accs[u % 4] = accs[u % 4] + hu * vzs[u]
                acc = (accs[0] + accs[1]) + (accs[2] + accs[3])
                out = out + jnp.where(j_ids != i2, acc, 0.0)
            return out

        pair = lax.fori_loop(0, N // 2, body, jnp.zeros((L,), jnp.float32))
        total = total + _C * pair

    tot = jnp.sum(0.125 * total)
    outv[...] = jnp.broadcast_to(tot, (L,))
    pltpu.sync_copy(outv, out.at[wid])


def kernel(pos, node_attrs, atom_emb, gamma, fc1_w, fc1_b, fc2_w, fc2_b,
           w_self, w_readout):
    ae_exp = (atom_emb[:, :, None] * w_readout[None, None, :, 0]).reshape(
        3, D_EMB * D_SCAL)
    fc2w1 = fc2_w[:D_EMB * D_SCAL, :]
    fc2b1 = fc2_b[:D_EMB * D_SCAL].reshape(D_EMB * D_SCAL, 1)
    fc1wT = fc1_w.T
    fc1b = fc1_b.reshape(1, 32)
    gamma2 = jnp.asarray(gamma, jnp.float32).reshape(1, 1)

    grid = (B // BPS,)
    full = lambda shape: pl.BlockSpec(shape, lambda b: (0,) * len(shape))
    hid, vt = pl.pallas_call(
        _tc_body,
        grid=grid,
        in_specs=[
            pl.BlockSpec((BPS, N, 3), lambda b: (b, 0, 0)),
            full((NUM_BASIS, 32)),
            full((1, 32)),
            full((1, 1)),
            full((3, D_EMB * D_SCAL)),
            full((D_EMB * D_SCAL, 32)),
            full((D_EMB * D_SCAL, 1)),
            full((D_EMB, D_SCAL)),
            full((D_SCAL, 1)),
            full((3, D_EMB)),
        ],
        out_specs=[
            pl.BlockSpec((BPS, D_EMB, N * N), lambda b: (b, 0, 0)),
            full((3, D_EMB + 1)),
        ],
        out_shape=[
            jax.ShapeDtypeStruct((B, D_EMB, N * N), jnp.float32),
            jax.ShapeDtypeStruct((3, D_EMB + 1), jnp.float32),
        ],
    )(pos, fc1wT, fc1b, gamma2, ae_exp, fc2w1, fc2b1, w_self, w_readout,
      atom_emb)

    mesh = plsc.VectorSubcoreMesh(core_axis_name="c", subcore_axis_name="s")
    sc = functools.partial(
        pl.kernel,
        mesh=mesh,
        compiler_params=pltpu.CompilerParams(needs_layout_passes=False),
        out_type=jax.ShapeDtypeStruct((B, L), jnp.float32),
        scratch_types=[
            pltpu.VMEM((D_EMB, N * N), jnp.float32),
            pltpu.VMEM((N, 3), jnp.float32),
            pltpu.VMEM((3, D_EMB + 1), jnp.float32),
            pltpu.VMEM((L,), jnp.float32),
        ],
    )(_sc_body)
    out = sc(hid, node_attrs, vt)
    return out[:, 0:1]
